# baseline (speedup 1.0000x reference)
import functools

import jax
import jax.numpy as jnp
from jax import lax
from jax.experimental import pallas as pl
from jax.experimental.pallas import tpu as pltpu
from jax.experimental.pallas import tpu_sc as plsc

_NUM_CORES = 2
_NUM_SUBCORES = 16
_NUM_WORKERS = _NUM_CORES * _NUM_SUBCORES
_CHUNK_ROWS = 32
_CHUNKS_PER_WORKER = 4
_NBUF = 3


def _read_copy(emb_hbm, buf, row):
    return pltpu.make_async_copy(
        emb_hbm.at[pl.ds(row, _CHUNK_ROWS), :], buf[0], buf[1]
    )


def _write_copy(out_hbm, buf, row):
    return pltpu.make_async_copy(
        buf[0], out_hbm.at[pl.ds(row, _CHUNK_ROWS), :], buf[2]
    )


def _sc_copy(emb_hbm, out_hbm, b0, b1, b2, r0, r1, r2, w0, w1, w2):
    sid = lax.axis_index("s")
    wid = sid * _NUM_CORES + lax.axis_index("c")
    base = wid * (_CHUNK_ROWS * _CHUNKS_PER_WORKER)
    bufs = ((b0.at[sid], r0, w0), (b1.at[sid], r1, w1), (b2.at[sid], r2, w2))
    for i in range(_NBUF):
        _read_copy(emb_hbm, bufs[i], base + i * _CHUNK_ROWS).start()
    for i in range(_CHUNKS_PER_WORKER):
        buf = bufs[i % _NBUF]
        row = base + i * _CHUNK_ROWS
        if i >= _NBUF:
            _write_copy(out_hbm, buf, base + (i - _NBUF) * _CHUNK_ROWS).wait()
            _read_copy(emb_hbm, buf, row).start()
        _read_copy(emb_hbm, buf, row).wait()
        _write_copy(out_hbm, buf, row).start()
    for i in range(max(0, _CHUNKS_PER_WORKER - _NBUF), _CHUNKS_PER_WORKER):
        buf = bufs[i % _NBUF]
        _write_copy(out_hbm, buf, base + i * _CHUNK_ROWS).wait()


def kernel(x, embedding):
    T = x.shape[1]
    F = embedding.shape[1]
    mesh = plsc.VectorSubcoreMesh(core_axis_name="c", subcore_axis_name="s")
    sc_copy = functools.partial(
        pl.kernel,
        mesh=mesh,
        out_type=jax.ShapeDtypeStruct((T, F), embedding.dtype),
        scratch_types=(
            [pltpu.VMEM_SHARED((_NUM_SUBCORES, _CHUNK_ROWS, F), embedding.dtype)]
            * _NBUF
            + [pltpu.SemaphoreType.DMA] * (2 * _NBUF)
        ),
    )(_sc_copy)
    return sc_copy(embedding)


# speedup vs baseline: 2.0048x; 2.0048x over previous
import jax
import jax.numpy as jnp
from jax.experimental import pallas as pl
from jax.experimental.pallas import tpu as pltpu

_CHUNK_ROWS = 512
_NCHUNK = 8
_NBUF = 4


def _ring_copy(emb_hbm, out_hbm, *scratch):
    bufs = scratch[:_NBUF]
    rsems = scratch[_NBUF : 2 * _NBUF]
    wsems = scratch[2 * _NBUF :]

    def read(i):
        return pltpu.make_async_copy(
            emb_hbm.at[pl.ds(i * _CHUNK_ROWS, _CHUNK_ROWS), :],
            bufs[i % _NBUF],
            rsems[i % _NBUF],
        )

    def write(i):
        return pltpu.make_async_copy(
            bufs[i % _NBUF],
            out_hbm.at[pl.ds(i * _CHUNK_ROWS, _CHUNK_ROWS), :],
            wsems[i % _NBUF],
        )

    for i in range(_NBUF):
        read(i).start()
    for i in range(_NCHUNK):
        if i >= _NBUF:
            write(i - _NBUF).wait()
            read(i).start()
        read(i).wait()
        write(i).start()
    for i in range(_NCHUNK - _NBUF, _NCHUNK):
        write(i).wait()


def kernel(x, embedding):
    T = x.shape[1]
    F = embedding.shape[1]
    return pl.pallas_call(
        _ring_copy,
        in_specs=[pl.BlockSpec(memory_space=pl.ANY)],
        out_specs=pl.BlockSpec(memory_space=pl.ANY),
        scratch_shapes=(
            [pltpu.VMEM((_CHUNK_ROWS, F), embedding.dtype)] * _NBUF
            + [pltpu.SemaphoreType.DMA] * (2 * _NBUF)
        ),
        out_shape=jax.ShapeDtypeStruct((T, F), embedding.dtype),
    )(embedding)


# speedup vs baseline: 2.7499x; 1.3717x over previous
import jax
import jax.numpy as jnp
from jax.experimental import pallas as pl
from jax.experimental.pallas import tpu as pltpu

_CHUNK_ROWS = 1024
_NCHUNK = 4
_NBUF = 4


def _ring_copy(emb_hbm, out_hbm, *scratch):
    bufs = scratch[:_NBUF]
    rsems = scratch[_NBUF : 2 * _NBUF]
    wsems = scratch[2 * _NBUF :]

    def read(i):
        return pltpu.make_async_copy(
            emb_hbm.at[pl.ds(i * _CHUNK_ROWS, _CHUNK_ROWS), :],
            bufs[i % _NBUF],
            rsems[i % _NBUF],
        )

    def write(i):
        return pltpu.make_async_copy(
            bufs[i % _NBUF],
            out_hbm.at[pl.ds(i * _CHUNK_ROWS, _CHUNK_ROWS), :],
            wsems[i % _NBUF],
        )

    for i in range(_NBUF):
        read(i).start()
    for i in range(_NCHUNK):
        if i >= _NBUF:
            write(i - _NBUF).wait()
            read(i).start()
        read(i).wait()
        write(i).start()
    for i in range(_NCHUNK - _NBUF, _NCHUNK):
        write(i).wait()


def kernel(x, embedding):
    T = x.shape[1]
    F = embedding.shape[1]
    return pl.pallas_call(
        _ring_copy,
        in_specs=[pl.BlockSpec(memory_space=pl.ANY)],
        out_specs=pl.BlockSpec(memory_space=pl.ANY),
        scratch_shapes=(
            [pltpu.VMEM((_CHUNK_ROWS, F), embedding.dtype)] * _NBUF
            + [pltpu.SemaphoreType.DMA] * (2 * _NBUF)
        ),
        out_shape=jax.ShapeDtypeStruct((T, F), embedding.dtype),
    )(embedding)


# speedup vs baseline: 2.7767x; 1.0098x over previous
import jax
import jax.numpy as jnp
from jax.experimental import pallas as pl


def _copy_block(emb_ref, out_ref):
    out_ref[...] = emb_ref[...]


def kernel(x, embedding):
    T = x.shape[1]
    F = embedding.shape[1]
    TILE = 2048
    return pl.pallas_call(
        _copy_block,
        grid=(T // TILE,),
        in_specs=[pl.BlockSpec((TILE, F), lambda i: (i, 0))],
        out_specs=pl.BlockSpec((TILE, F), lambda i: (i, 0)),
        out_shape=jax.ShapeDtypeStruct((T, F), embedding.dtype),
    )(embedding)
